# baseline (speedup 1.0000x reference)
import functools

import jax
import jax.numpy as jnp
from jax import lax
from jax.experimental import pallas as pl
from jax.experimental.pallas import tpu as pltpu
from jax.experimental.pallas import tpu_sc as plsc


def kernel(input_tensor, table):
    B, H = input_tensor.shape
    V, D = table.shape

    info = plsc.get_sparse_core_info()
    NC, NS = info.num_cores, info.num_subcores
    NW = NC * NS

    assert B % NW == 0
    b_per_w = B // NW

    idx = input_tensor.astype(jnp.int32)

    mesh = plsc.VectorSubcoreMesh(core_axis_name="c", subcore_axis_name="s")

    @functools.partial(
        pl.kernel,
        out_type=jax.ShapeDtypeStruct((B, H, D), jnp.float32),
        mesh=mesh,
        scratch_types=[
            pltpu.VMEM((b_per_w, H), jnp.int32),
            pltpu.VMEM((H, D), jnp.float32),
            pltpu.SemaphoreType.DMA,
        ],
        compiler_params=pltpu.CompilerParams(use_tc_tiling_on_sc=False),
    )
    def emb(idx_hbm, table_hbm, out_hbm, idx_v, rows_v, sem):
        wid = lax.axis_index("s") * NC + lax.axis_index("c")
        b0 = wid * b_per_w
        pltpu.sync_copy(idx_hbm.at[pl.ds(b0, b_per_w)], idx_v)

        def body(b, carry):
            pltpu.async_copy(table_hbm.at[idx_v.at[b]], rows_v, sem).wait()
            pltpu.sync_copy(rows_v, out_hbm.at[b0 + b])
            return carry

        lax.fori_loop(0, b_per_w, body, 0)

    out = emb(idx, table)
    return out


# speedup vs baseline: 2.0365x; 2.0365x over previous
import functools

import jax
import jax.numpy as jnp
from jax import lax
from jax.experimental import pallas as pl
from jax.experimental.pallas import tpu as pltpu
from jax.experimental.pallas import tpu_sc as plsc


def kernel(input_tensor, table):
    B, H = input_tensor.shape
    V, D = table.shape
    N = B * H
    Hp = (H + 7) // 8 * 8
    Dp = 128

    info = plsc.get_sparse_core_info()
    NC, NS = info.num_cores, info.num_subcores
    NW = NC * NS

    K = 2 * H
    assert N % (NW * K) == 0
    n_per_w = N // NW
    n_ck = n_per_w // K
    b_per_w = B // NW

    idx = input_tensor.reshape(N // K, K).astype(jnp.int32)

    mesh = plsc.VectorSubcoreMesh(core_axis_name="c", subcore_axis_name="s")

    NBUF = 4

    @functools.partial(
        pl.kernel,
        out_type=jax.ShapeDtypeStruct((B, Hp, Dp), jnp.float32),
        mesh=mesh,
        scratch_types=[
            pltpu.VMEM((n_ck, K), jnp.int32),
            pltpu.VMEM((NBUF, K, D), jnp.float32),
            pltpu.SemaphoreType.DMA,
            [pltpu.SemaphoreType.DMA] * NBUF,
            [pltpu.SemaphoreType.DMA] * NBUF,
        ],
        compiler_params=pltpu.CompilerParams(use_tc_tiling_on_sc=False),
    )
    def emb(idx_hbm, table_hbm, out_hbm, idx_v, rows_v, isem, gsems, wsems):
        wid = lax.axis_index("s") * NC + lax.axis_index("c")
        b0 = wid * b_per_w
        pltpu.async_copy(idx_hbm.at[pl.ds(wid * n_ck, n_ck)], idx_v, isem).wait()

        def gstart(c, j):
            pltpu.async_copy(
                table_hbm.at[idx_v.at[c]], rows_v.at[j], gsems[j]
            )

        def gwait(c, j):
            pltpu.make_async_copy(
                table_hbm.at[idx_v.at[c]], rows_v.at[j], gsems[j]
            ).wait()

        def wstart(c, j):
            b = b0 + 2 * c
            pltpu.async_copy(
                rows_v.at[j, pl.ds(0, H)],
                out_hbm.at[b, pl.ds(0, H), pl.ds(0, D)],
                wsems[j],
            )
            pltpu.async_copy(
                rows_v.at[j, pl.ds(H, H)],
                out_hbm.at[b + 1, pl.ds(0, H), pl.ds(0, D)],
                wsems[j],
            )

        def wwait(c, j):
            b = b0 + 2 * c
            pltpu.make_async_copy(
                rows_v.at[j, pl.ds(0, H)],
                out_hbm.at[b, pl.ds(0, H), pl.ds(0, D)],
                wsems[j],
            ).wait()
            pltpu.make_async_copy(
                rows_v.at[j, pl.ds(H, H)],
                out_hbm.at[b + 1, pl.ds(0, H), pl.ds(0, D)],
                wsems[j],
            ).wait()

        gstart(0, 0)
        gstart(1, 1)

        def body(gi, carry):
            base = gi * NBUF
            for j in range(NBUF):
                c = base + j
                jj = (j + 2) % NBUF

                @pl.when(c >= 2)
                def _():
                    wwait(c - 2, jj)

                @pl.when(c + 2 < n_ck)
                def _():
                    gstart(c + 2, jj)

                gwait(c, j)
                wstart(c, j)
            return carry

        lax.fori_loop(0, n_ck // NBUF, body, 0)
        wwait(n_ck - 2, (n_ck - 2) % NBUF)
        wwait(n_ck - 1, (n_ck - 1) % NBUF)

    out_p = emb(idx, table)
    return out_p[:, :H, :D]
